# baseline (speedup 1.0000x reference)
import jax
import jax.numpy as jnp
from jax.experimental import pallas as pl
from jax.experimental.pallas import tpu as pltpu

_SCORE_WEIGHT = 0.5
_MAX_ITER = 20
_PROJ_ITER = 5
_RELAX_LR = 0.1
_EPS = 1e-8

_HB = 48
_VMEM_LIMIT = 52 * 1024 * 1024


def _fused_body(a_ref, b_ref, pf_ref, tf_ref, ps_ref,
                out_ref, ms_ref, ds_ref,
                b8_scr, inter_scr, asum_scr, binx_scr):
    ph = pl.program_id(0)
    j = pl.program_id(1)
    o, hb, w = a_ref.shape
    p, ch = b_ref.shape
    half = p // 2

    @pl.when((ph == 0) & (j == 0))
    def _():
        inter_scr[...] = jnp.zeros_like(inter_scr)
        asum_scr[...] = jnp.zeros_like(asum_scr)

    @pl.when(ph == 0)
    def _phase0():
        a = a_ref[...].reshape(o, hb * w)
        lhs = jnp.concatenate(
            [a.astype(jnp.bfloat16), jnp.ones((8, ch), jnp.bfloat16)],
            axis=0)
        b8 = b_ref[...]
        bbf = b8.astype(jnp.bfloat16)
        acc = jax.lax.dot_general(lhs, bbf, (((1,), (1,)), ((), ())),
                                  preferred_element_type=jnp.float32)
        inter_scr[...] += acc
        asum_scr[...] += jnp.sum(a, axis=1, keepdims=True)
        b8_scr[j, :half, :] = b8[:half, :]
        b8_scr[j, half:, :] = b8[half:, :]

    @pl.when((ph == 1) & (j == 0))
    def _solve():
        inter = inter_scr[:o, :]
        bsum = inter_scr[o:o + 1, :]
        asum = asum_scr[...]
        union = asum + bsum - inter
        iou = inter / (union + _EPS)

        pf = pf_ref[...]
        kf = pf / (jnp.sqrt(jnp.sum(pf * pf, axis=1, keepdims=True)) + _EPS)
        tf = tf_ref[...]
        qn = jnp.sqrt(jnp.sum(tf * tf, axis=2, keepdims=True)) + _EPS
        qf = tf / qn
        qsum = jnp.sum(qf, axis=0)
        feature_sim = jax.lax.dot_general(
            qsum, kf, (((1,), (1,)), ((), ())),
            preferred_element_type=jnp.float32) / tf_ref.shape[0]

        sim = feature_sim * (1.0 - _SCORE_WEIGHT) + iou * _SCORE_WEIGHT
        cost = -sim

        x0 = jnp.full((o, p), 1.0 / p, dtype=jnp.float32)

        def proj_body(_, x):
            x = jnp.clip(x, 0.0, 1.0)
            return x / (jnp.sum(x, axis=1, keepdims=True) + _EPS)

        def outer(_, carry):
            x, s = carry
            xn = jax.lax.fori_loop(0, _PROJ_ITER, proj_body,
                                   x - _RELAX_LR * cost)
            return xn, s + xn

        _, s = jax.lax.fori_loop(
            0, _MAX_ITER, outer, (x0, jnp.zeros((o, p), dtype=jnp.float32)))
        ridx = s / jnp.float32(_MAX_ITER)

        logic = (ridx > 0.01).astype(jnp.float32)
        binx = ridx * logic
        binx_scr[...] = binx
        ms_ref[...] = jnp.max(jnp.clip(ridx, 0.0, 1.0) * sim, axis=1,
                              keepdims=True)
        ds_ref[...] = jnp.sum(ps_ref[...] * binx, axis=1, keepdims=True)

    @pl.when(ph == 1)
    def _phase1():
        x = binx_scr[...]
        xh = x.astype(jnp.bfloat16)
        xl = (x - xh.astype(jnp.float32)).astype(jnp.bfloat16)
        xs = jnp.concatenate([xh, xl], axis=0)
        bbf = b8_scr[j].astype(jnp.bfloat16)
        dn = (((1,), (0,)), ((), ()))
        both = jax.lax.dot_general(xs, bbf, dn,
                                   preferred_element_type=jnp.float32)
        flat = both[:o, :] + both[o:, :]
        out_ref[...] = flat.reshape(o, hb, w)


def kernel(proposed_feature, proposed_mask, template_feature,
           mask_last_occurence, proposal_score):
    p, d = proposed_feature.shape
    o = mask_last_occurence.shape[0]
    t = template_feature.shape[0]
    h, w = proposed_mask.shape[1], proposed_mask.shape[2]
    hw = h * w
    ch = _HB * w
    nc = h // _HB

    b2 = jax.lax.optimization_barrier(
        proposed_mask.reshape(p, hw).astype(jnp.int8))

    outmask, ms, ds = pl.pallas_call(
        _fused_body,
        grid=(2, nc),
        in_specs=[
            pl.BlockSpec((o, _HB, w), lambda ph, j: (0, j * (1 - ph), 0)),
            pl.BlockSpec((p, ch), lambda ph, j: (0, j * (1 - ph))),
            pl.BlockSpec((p, d), lambda ph, j: (0, 0)),
            pl.BlockSpec((t, o, d), lambda ph, j: (0, 0, 0)),
            pl.BlockSpec((1, p), lambda ph, j: (0, 0)),
        ],
        out_specs=[
            pl.BlockSpec((o, _HB, w), lambda ph, j: (0, j * ph, 0)),
            pl.BlockSpec((o, 1), lambda ph, j: (0, 0)),
            pl.BlockSpec((o, 1), lambda ph, j: (0, 0)),
        ],
        out_shape=[
            jax.ShapeDtypeStruct((o, h, w), jnp.float32),
            jax.ShapeDtypeStruct((o, 1), jnp.float32),
            jax.ShapeDtypeStruct((o, 1), jnp.float32),
        ],
        scratch_shapes=[
            pltpu.VMEM((nc, p, ch), jnp.int8),
            pltpu.VMEM((o + 8, p), jnp.float32),
            pltpu.VMEM((o, 1), jnp.float32),
            pltpu.VMEM((o, p), jnp.float32),
        ],
        compiler_params=pltpu.CompilerParams(
            dimension_semantics=("arbitrary", "arbitrary"),
            vmem_limit_bytes=_VMEM_LIMIT),
        name="match_model_fused",
    )(mask_last_occurence, b2, proposed_feature, template_feature,
      proposal_score.reshape(1, p))

    return (outmask, ms.reshape(o), ds.reshape(o))
